# baseline (speedup 1.0000x reference)
import functools

import jax
import jax.numpy as jnp
from jax import lax
from jax.experimental import pallas as pl
from jax.experimental.pallas import tpu as pltpu
from jax.experimental.pallas import tpu_sc as plsc

TOKEN_DIM = 1024
HID = 32


def _sc_gather_to_slot(table, idx):
    B = idx.shape[0]
    D = table.shape[1]
    info = plsc.get_sparse_core_info()
    nw = info.num_cores * info.num_subcores
    b_per_w = B // nw
    CH = 32
    n_ch = b_per_w // CH
    mesh = plsc.VectorSubcoreMesh(core_axis_name="c", subcore_axis_name="s")

    @functools.partial(
        pl.kernel,
        mesh=mesh,
        out_type=jax.ShapeDtypeStruct((B, 5 * D), jnp.float32),
        scratch_types=[
            pltpu.VMEM((b_per_w,), jnp.int32),
            pltpu.VMEM((CH, D), jnp.float32),
            pltpu.VMEM((CH, D), jnp.float32),
            pltpu.SemaphoreType.DMA,
            pltpu.SemaphoreType.DMA,
        ],
    )
    def k(table_hbm, idx_hbm, out_hbm, idx_v, rows0, rows1, sem0, sem1):
        wid = lax.axis_index("s") * info.num_cores + lax.axis_index("c")
        base = wid * b_per_w
        pltpu.sync_copy(idx_hbm.at[pl.ds(base, b_per_w)], idx_v)
        bufs = (rows0, rows1)
        sems = (sem0, sem1)

        def gather(c):
            return pltpu.make_async_copy(
                table_hbm.at[idx_v.at[pl.ds(c * CH, CH)]],
                bufs[c % 2], sems[c % 2])

        gather(0).start()
        for c in range(n_ch):
            if c + 1 < n_ch:
                gather(c + 1).start()
            gather(c).wait()
            pltpu.sync_copy(
                bufs[c % 2],
                out_hbm.at[pl.ds(base + c * CH, CH), pl.ds(2 * D, D)])

    return k(table, idx)


def _tc_body(buf_ref, td_ref, nv_ref, sm_ref, vm_ref,
             dW1, db1, dW2, db2, vW1, vb1, vW2, vb2,
             tst, cpf, vpf, out_ref):
    del buf_ref
    D = TOKEN_DIM
    blk = td_ref.shape[0]
    j = pl.program_id(1)

    td = td_ref[...]
    h_t = jnp.tanh(td * dW1[...] + db1[...])
    emb_t = jnp.dot(h_t, dW2[...],
                    preferred_element_type=jnp.float32) + db2[...]
    nv = nv_ref[...]
    h_v = jnp.tanh(nv * vW1[...] + vb1[...])
    emb_v = jnp.dot(h_v, vW2[...],
                    preferred_element_type=jnp.float32) + vb2[...]

    tmask = (sm_ref[...] > 0.0) & (td != 0.0)
    vmask = vm_ref[...] > 0.0

    ts_row = jnp.broadcast_to(tst[...], (blk, D))
    cp_row = jnp.broadcast_to(cpf[...], (blk, D))
    vp_row = jnp.broadcast_to(vpf[...], (blk, D))
    ts_val = jnp.where(tmask, emb_t, ts_row)
    va_val = jnp.where(vmask, emb_v, vp_row)

    res = jnp.where(j == 0, ts_val,
                    jnp.where(j == 1, cp_row,
                              jnp.where(j == 2, vp_row, va_val)))
    out_ref[...] = res


def _tc_assemble(buf, td, nv, sm, vm,
                 dW1, db1, dW2, db2, vW1, vb1, vW2, vb2,
                 tst, cpf, vpf):
    N = td.shape[0]
    D = TOKEN_DIM
    BLK = 512
    grid = (N // BLK, 4)

    col = lambda i, j: (i, 0)
    rep = lambda i, j: (0, 0)
    slot = lambda i, j: (i, j + (j >= 2).astype(jnp.int32))
    specs = [
        pl.BlockSpec(memory_space=pltpu.MemorySpace.HBM),
        pl.BlockSpec((BLK, 1), col),
        pl.BlockSpec((BLK, 1), col),
        pl.BlockSpec((BLK, 1), col),
        pl.BlockSpec((BLK, 1), col),
        pl.BlockSpec((1, HID), rep),
        pl.BlockSpec((1, HID), rep),
        pl.BlockSpec((HID, D), rep),
        pl.BlockSpec((1, D), rep),
        pl.BlockSpec((1, HID), rep),
        pl.BlockSpec((1, HID), rep),
        pl.BlockSpec((HID, D), rep),
        pl.BlockSpec((1, D), rep),
        pl.BlockSpec((1, D), rep),
        pl.BlockSpec((1, D), rep),
        pl.BlockSpec((1, D), rep),
    ]
    return pl.pallas_call(
        _tc_body,
        grid=grid,
        in_specs=specs,
        out_specs=pl.BlockSpec((BLK, D), slot),
        out_shape=jax.ShapeDtypeStruct((N, 5 * D), jnp.float32),
        input_output_aliases={0: 0},
    )(buf, td, nv, sm, vm,
      dW1, db1, dW2, db2, vW1, vb1, vW2, vb2, tst, cpf, vpf)


def kernel(static_mask, code, numerical_value, time_delta_days,
           numerical_value_mask, mask, code_table,
           date_W1, date_b1, date_W2, date_b2,
           val_W1, val_b1, val_W2, val_b2,
           ts_token, code_prefix, val_prefix):
    N = code.shape[0]
    buf = _sc_gather_to_slot(code_table, code.astype(jnp.int32))

    col = lambda a: a.astype(jnp.float32).reshape(N, 1)
    row = lambda a: a.reshape(1, -1)
    return _tc_assemble(
        buf, col(time_delta_days), col(numerical_value),
        col(static_mask), col(numerical_value_mask),
        date_W1, row(date_b1), date_W2, row(date_b2),
        val_W1, row(val_b1), val_W2, row(val_b2),
        row(ts_token), row(code_prefix), row(val_prefix))


# speedup vs baseline: 1.2153x; 1.2153x over previous
import functools

import jax
import jax.numpy as jnp
from jax import lax
from jax.experimental import pallas as pl
from jax.experimental.pallas import tpu as pltpu
from jax.experimental.pallas import tpu_sc as plsc

TOKEN_DIM = 1024
HID = 32


def _sc_gather(table, idx):
    B = idx.shape[0]
    D = table.shape[1]
    info = plsc.get_sparse_core_info()
    nw = info.num_cores * info.num_subcores
    b_per_w = B // nw
    CH = 32
    n_ch = b_per_w // CH
    mesh = plsc.VectorSubcoreMesh(core_axis_name="c", subcore_axis_name="s")

    @functools.partial(
        pl.kernel,
        mesh=mesh,
        out_type=jax.ShapeDtypeStruct((B, D), jnp.float32),
        scratch_types=[
            pltpu.VMEM((b_per_w,), jnp.int32),
            pltpu.VMEM((CH, D), jnp.float32),
            pltpu.VMEM((CH, D), jnp.float32),
            pltpu.SemaphoreType.DMA,
            pltpu.SemaphoreType.DMA,
        ],
    )
    def k(table_hbm, idx_hbm, out_hbm, idx_v, rows0, rows1, sem0, sem1):
        wid = lax.axis_index("s") * info.num_cores + lax.axis_index("c")
        base = wid * b_per_w
        pltpu.sync_copy(idx_hbm.at[pl.ds(base, b_per_w)], idx_v)
        bufs = (rows0, rows1)
        sems = (sem0, sem1)

        def gather(c):
            return pltpu.make_async_copy(
                table_hbm.at[idx_v.at[pl.ds(c * CH, CH)]],
                bufs[c % 2], sems[c % 2])

        gather(0).start()
        for c in range(n_ch):
            if c + 1 < n_ch:
                gather(c + 1).start()
            gather(c).wait()
            pltpu.sync_copy(bufs[c % 2], out_hbm.at[pl.ds(base + c * CH, CH)])

    return k(table, idx)


def _tc_body(td_ref, nv_ref, sm_ref, vm_ref, g_ref,
             dW1, db1, dW2, db2, vW1, vb1, vW2, vb2,
             tst, cpf, vpf, out_ref):
    D = TOKEN_DIM
    blk = td_ref.shape[0]

    td = td_ref[...]
    h_t = jnp.tanh(td * dW1[...] + db1[...])
    emb_t = jnp.dot(h_t, dW2[...],
                    preferred_element_type=jnp.float32) + db2[...]

    nv = nv_ref[...]
    h_v = jnp.tanh(nv * vW1[...] + vb1[...])
    emb_v = jnp.dot(h_v, vW2[...],
                    preferred_element_type=jnp.float32) + vb2[...]

    tmask = (sm_ref[...] > 0.0) & (td != 0.0)
    vmask = vm_ref[...] > 0.0

    ts_row = jnp.broadcast_to(tst[...], (blk, D))
    vp_row = jnp.broadcast_to(vpf[...], (blk, D))

    out_ref[:, 0 * D:1 * D] = jnp.where(tmask, emb_t, ts_row)
    out_ref[:, 1 * D:2 * D] = jnp.broadcast_to(cpf[...], (blk, D))
    out_ref[:, 2 * D:3 * D] = g_ref[...]
    out_ref[:, 3 * D:4 * D] = vp_row
    out_ref[:, 4 * D:5 * D] = jnp.where(vmask, emb_v, vp_row)


def _tc_assemble(td, nv, sm, vm, g,
                 dW1, db1, dW2, db2, vW1, vb1, vW2, vb2,
                 tst, cpf, vpf):
    N = td.shape[0]
    D = TOKEN_DIM
    BLK = 1024
    grid = (N // BLK,)

    col = lambda i: (i, 0)
    rep = lambda i: (0, 0)
    specs = [
        pl.BlockSpec((BLK, 1), col),
        pl.BlockSpec((BLK, 1), col),
        pl.BlockSpec((BLK, 1), col),
        pl.BlockSpec((BLK, 1), col),
        pl.BlockSpec((BLK, D), col),
        pl.BlockSpec((1, HID), rep),
        pl.BlockSpec((1, HID), rep),
        pl.BlockSpec((HID, D), rep),
        pl.BlockSpec((1, D), rep),
        pl.BlockSpec((1, HID), rep),
        pl.BlockSpec((1, HID), rep),
        pl.BlockSpec((HID, D), rep),
        pl.BlockSpec((1, D), rep),
        pl.BlockSpec((1, D), rep),
        pl.BlockSpec((1, D), rep),
        pl.BlockSpec((1, D), rep),
    ]
    return pl.pallas_call(
        _tc_body,
        grid=grid,
        in_specs=specs,
        out_specs=pl.BlockSpec((BLK, 5 * D), col),
        out_shape=jax.ShapeDtypeStruct((N, 5 * D), jnp.float32),
    )(td, nv, sm, vm, g,
      dW1, db1, dW2, db2, vW1, vb1, vW2, vb2, tst, cpf, vpf)


def kernel(static_mask, code, numerical_value, time_delta_days,
           numerical_value_mask, mask, code_table,
           date_W1, date_b1, date_W2, date_b2,
           val_W1, val_b1, val_W2, val_b2,
           ts_token, code_prefix, val_prefix):
    N = code.shape[0]
    g = _sc_gather(code_table, code.astype(jnp.int32))

    col = lambda a: a.astype(jnp.float32).reshape(N, 1)
    row = lambda a: a.reshape(1, -1)
    return _tc_assemble(
        col(time_delta_days), col(numerical_value),
        col(static_mask), col(numerical_value_mask), g,
        date_W1, row(date_b1), date_W2, row(date_b2),
        val_W1, row(val_b1), val_W2, row(val_b2),
        row(ts_token), row(code_prefix), row(val_prefix))
